# initial kernel scaffold (unmeasured)
import jax
import jax.numpy as jnp
from jax import lax
from jax.experimental import pallas as pl
from jax.experimental.pallas import tpu as pltpu

N_DEV = 4
N_TOK = 2048
D_MODEL = 1024
N_LOCAL = 8
CAPACITY = 51
CHUNK = N_TOK // N_DEV
N_HOP = N_DEV - 1


def _body(x_ref, mask_ref, ew_ref, out_ref, comm_ref, send_sems, rs_sems, ag_sems):
    k = pl.program_id(0)

    contrib = jnp.dot(
        x_ref[...] * mask_ref[...],
        ew_ref[0],
        preferred_element_type=jnp.float32,
    )

    @pl.when(k == 0)
    def _():
        out_ref[...] = contrib

    @pl.when(k > 0)
    def _():
        out_ref[...] += contrib

    @pl.when(k == N_LOCAL - 1)
    def _():
        my = lax.axis_index("i")
        left = lax.rem(my + N_DEV - 1, N_DEV)
        right = lax.rem(my + 1, N_DEV)

        barrier_sem = pltpu.get_barrier_semaphore()
        for nbr in (left, right):
            pl.semaphore_signal(
                barrier_sem, inc=1,
                device_id=(nbr,), device_id_type=pl.DeviceIdType.MESH,
            )
        pl.semaphore_wait(barrier_sem, 2)

        for h in range(N_HOP):
            s_idx = lax.rem(my - h + N_DEV, N_DEV)
            r_idx = lax.rem(my - h - 1 + N_DEV, N_DEV)
            rdma = pltpu.make_async_remote_copy(
                src_ref=out_ref.at[pl.ds(s_idx * CHUNK, CHUNK), :],
                dst_ref=comm_ref.at[h],
                send_sem=send_sems.at[h],
                recv_sem=rs_sems.at[h],
                device_id=(right,),
                device_id_type=pl.DeviceIdType.MESH,
            )
            rdma.start()
            rdma.wait()
            out_ref[pl.ds(r_idx * CHUNK, CHUNK), :] += comm_ref[h]

        for h in range(N_HOP):
            s_idx = lax.rem(my + 1 - h + N_DEV, N_DEV)
            rdma = pltpu.make_async_remote_copy(
                src_ref=out_ref.at[pl.ds(s_idx * CHUNK, CHUNK), :],
                dst_ref=out_ref.at[pl.ds(s_idx * CHUNK, CHUNK), :],
                send_sem=send_sems.at[N_HOP + h],
                recv_sem=ag_sems.at[h],
                device_id=(right,),
                device_id_type=pl.DeviceIdType.MESH,
            )
            rdma.start()
            rdma.wait()


def kernel(x, router_W, route_idx, expert_W):
    del router_W

    my = lax.axis_index("i")
    e_ids = my * N_LOCAL + jnp.arange(N_LOCAL, dtype=jnp.int32)
    onehot = (route_idx == e_ids[None, :]).astype(jnp.float32)
    pos = jnp.cumsum(onehot, axis=0)
    mask = onehot * (pos <= CAPACITY)

    return pl.pallas_call(
        _body,
        grid=(N_LOCAL,),
        in_specs=[
            pl.BlockSpec((N_TOK, D_MODEL), lambda k: (0, 0)),
            pl.BlockSpec((N_TOK, 1), lambda k: (0, k)),
            pl.BlockSpec((1, D_MODEL, D_MODEL), lambda k: (k, 0, 0)),
        ],
        out_specs=pl.BlockSpec((N_TOK, D_MODEL), lambda k: (0, 0)),
        out_shape=jax.ShapeDtypeStruct((N_TOK, D_MODEL), jnp.float32),
        scratch_shapes=[
            pltpu.VMEM((N_HOP, CHUNK, D_MODEL), jnp.float32),
            pltpu.SemaphoreType.DMA((2 * N_HOP,)),
            pltpu.SemaphoreType.DMA((N_HOP,)),
            pltpu.SemaphoreType.DMA((N_HOP,)),
        ],
        compiler_params=pltpu.CompilerParams(
            collective_id=0,
            dimension_semantics=("arbitrary",),
        ),
    )(x, mask, expert_W)


# baseline (device time: 206057 ns/iter reference)
import jax
import jax.numpy as jnp
from jax import lax
from jax.experimental import pallas as pl
from jax.experimental.pallas import tpu as pltpu

N_DEV = 4
N_TOK = 2048
D_MODEL = 1024
N_LOCAL = 8
CAPACITY = 51
CHUNK = N_TOK // N_DEV
N_HOP = N_DEV - 1


def _body(x_ref, mask_ref, ew_ref, out_ref, comm_ref, send_sems, rs_sems, ag_sems):
    k = pl.program_id(0)

    col = lax.broadcasted_iota(jnp.int32, (1, N_LOCAL), 1) == k
    mask_k = jnp.sum(mask_ref[...] * col.astype(jnp.float32), axis=1, keepdims=True)
    contrib = jnp.dot(
        x_ref[...] * mask_k,
        ew_ref[0],
        preferred_element_type=jnp.float32,
    )

    @pl.when(k == 0)
    def _():
        out_ref[...] = contrib

    @pl.when(k > 0)
    def _():
        out_ref[...] += contrib

    @pl.when(k == N_LOCAL - 1)
    def _():
        my = lax.axis_index("i")
        left = lax.rem(my + N_DEV - 1, N_DEV)
        right = lax.rem(my + 1, N_DEV)

        barrier_sem = pltpu.get_barrier_semaphore()
        for nbr in (left, right):
            pl.semaphore_signal(
                barrier_sem, inc=1,
                device_id=(nbr,), device_id_type=pl.DeviceIdType.MESH,
            )
        pl.semaphore_wait(barrier_sem, 2)

        for h in range(N_HOP):
            s_idx = lax.rem(my - h + N_DEV, N_DEV)
            r_idx = lax.rem(my - h - 1 + N_DEV, N_DEV)
            rdma = pltpu.make_async_remote_copy(
                src_ref=out_ref.at[pl.ds(s_idx * CHUNK, CHUNK), :],
                dst_ref=comm_ref.at[h],
                send_sem=send_sems.at[h],
                recv_sem=rs_sems.at[h],
                device_id=(right,),
                device_id_type=pl.DeviceIdType.MESH,
            )
            rdma.start()
            rdma.wait()
            out_ref[pl.ds(r_idx * CHUNK, CHUNK), :] += comm_ref[h]

        for h in range(N_HOP):
            s_idx = lax.rem(my + 1 - h + N_DEV, N_DEV)
            rdma = pltpu.make_async_remote_copy(
                src_ref=out_ref.at[pl.ds(s_idx * CHUNK, CHUNK), :],
                dst_ref=out_ref.at[pl.ds(s_idx * CHUNK, CHUNK), :],
                send_sem=send_sems.at[N_HOP + h],
                recv_sem=ag_sems.at[h],
                device_id=(right,),
                device_id_type=pl.DeviceIdType.MESH,
            )
            rdma.start()
            rdma.wait()


def kernel(x, router_W, route_idx, expert_W):
    del router_W

    my = lax.axis_index("i")
    e_ids = my * N_LOCAL + jnp.arange(N_LOCAL, dtype=jnp.int32)
    onehot = (route_idx == e_ids[None, :]).astype(jnp.float32)
    pos = jnp.cumsum(onehot, axis=0)
    mask = onehot * (pos <= CAPACITY)

    return pl.pallas_call(
        _body,
        grid=(N_LOCAL,),
        in_specs=[
            pl.BlockSpec((N_TOK, D_MODEL), lambda k: (0, 0)),
            pl.BlockSpec((N_TOK, N_LOCAL), lambda k: (0, 0)),
            pl.BlockSpec((1, D_MODEL, D_MODEL), lambda k: (k, 0, 0)),
        ],
        out_specs=pl.BlockSpec((N_TOK, D_MODEL), lambda k: (0, 0)),
        out_shape=jax.ShapeDtypeStruct((N_TOK, D_MODEL), jnp.float32),
        scratch_shapes=[
            pltpu.VMEM((N_HOP, CHUNK, D_MODEL), jnp.float32),
            pltpu.SemaphoreType.DMA((2 * N_HOP,)),
            pltpu.SemaphoreType.DMA((N_HOP,)),
            pltpu.SemaphoreType.DMA((N_HOP,)),
        ],
        compiler_params=pltpu.CompilerParams(
            collective_id=0,
            dimension_semantics=("arbitrary",),
        ),
    )(x, mask, expert_W)


# device time: 80502 ns/iter; 2.5597x vs baseline; 2.5597x over previous
import jax
import jax.numpy as jnp
from jax import lax
from jax.experimental import pallas as pl
from jax.experimental.pallas import tpu as pltpu

N_DEV = 4
N_TOK = 2048
D_MODEL = 1024
N_EXP = 32
N_LOCAL = 8
CAPACITY = 51
CAP_PAD = 64
BLK = N_LOCAL * CAP_PAD
HALF = BLK // 2
N_HOP = N_DEV - 1
SENTINEL = 1 << 20


def _body(x_ref, slot_row_ref, slot_col_ref, ew_ref, out_ref,
          blocks_ref, cw_send, ccw_send, cw_recv, ccw_recv):
    k = pl.program_id(0)
    my = lax.axis_index("i")

    base = my * BLK + k * CAP_PAD
    iota_s = lax.broadcasted_iota(jnp.int32, (CAP_PAD, 1), 0)
    dt = (slot_row_ref[...] == iota_s + base).astype(jnp.float32)
    compact_x = jnp.dot(dt, x_ref[...], preferred_element_type=jnp.float32)
    y = jnp.dot(compact_x, ew_ref[0], preferred_element_type=jnp.float32)
    blocks_ref[my, pl.ds(k * CAP_PAD, CAP_PAD), :] = y

    @pl.when(k == N_LOCAL - 1)
    def _():
        left = lax.rem(my + N_DEV - 1, N_DEV)
        right = lax.rem(my + 1, N_DEV)

        barrier_sem = pltpu.get_barrier_semaphore()
        for nbr in (left, right):
            pl.semaphore_signal(
                barrier_sem, inc=1,
                device_id=(nbr,), device_id_type=pl.DeviceIdType.MESH,
            )
        pl.semaphore_wait(barrier_sem, 2)

        def scatter(blk_idx, half, init):
            off = blk_idx * BLK + half * HALF
            iota_c = lax.broadcasted_iota(jnp.int32, (N_TOK, HALF), 1)
            d = (slot_col_ref[...] == iota_c + off).astype(jnp.float32)
            contrib = jnp.dot(
                d,
                blocks_ref[blk_idx, pl.ds(half * HALF, HALF), :],
                preferred_element_type=jnp.float32,
            )
            if init:
                out_ref[...] = contrib
            else:
                out_ref[...] += contrib

        for h in range(N_HOP):
            cw_idx = lax.rem(my - h + N_DEV, N_DEV)
            ccw_idx = lax.rem(my + h, N_DEV)
            cw = pltpu.make_async_remote_copy(
                src_ref=blocks_ref.at[cw_idx, pl.ds(0, HALF), :],
                dst_ref=blocks_ref.at[cw_idx, pl.ds(0, HALF), :],
                send_sem=cw_send.at[h],
                recv_sem=cw_recv.at[h],
                device_id=(right,),
                device_id_type=pl.DeviceIdType.MESH,
            )
            ccw = pltpu.make_async_remote_copy(
                src_ref=blocks_ref.at[ccw_idx, pl.ds(HALF, HALF), :],
                dst_ref=blocks_ref.at[ccw_idx, pl.ds(HALF, HALF), :],
                send_sem=ccw_send.at[h],
                recv_sem=ccw_recv.at[h],
                device_id=(left,),
                device_id_type=pl.DeviceIdType.MESH,
            )
            cw.start()
            ccw.start()
            if h == 0:
                scatter(my, 0, init=True)
                scatter(my, 1, init=False)
            else:
                scatter(lax.rem(my - h + N_DEV, N_DEV), 0, init=False)
                scatter(lax.rem(my + h, N_DEV), 1, init=False)
            cw.wait()
            ccw.wait()

        scatter(lax.rem(my + 1, N_DEV), 0, init=False)
        scatter(lax.rem(my - 1 + N_DEV, N_DEV), 1, init=False)


def kernel(x, router_W, route_idx, expert_W):
    del router_W

    e_ids = jnp.arange(N_EXP, dtype=jnp.int32)
    onehot = route_idx == e_ids[None, :]
    pos = jnp.cumsum(onehot.astype(jnp.int32), axis=0)
    pos_t = jnp.sum(pos * onehot, axis=1)
    kept = jnp.any(onehot & (pos <= CAPACITY), axis=1)
    slot = jnp.where(
        kept, route_idx[:, 0] * CAP_PAD + pos_t - 1, SENTINEL
    ).astype(jnp.int32)

    return pl.pallas_call(
        _body,
        grid=(N_LOCAL,),
        in_specs=[
            pl.BlockSpec((N_TOK, D_MODEL), lambda k: (0, 0)),
            pl.BlockSpec((1, N_TOK), lambda k: (0, 0)),
            pl.BlockSpec((N_TOK, 1), lambda k: (0, 0)),
            pl.BlockSpec((1, D_MODEL, D_MODEL), lambda k: (k, 0, 0)),
        ],
        out_specs=pl.BlockSpec((N_TOK, D_MODEL), lambda k: (0, 0)),
        out_shape=jax.ShapeDtypeStruct((N_TOK, D_MODEL), jnp.float32),
        scratch_shapes=[
            pltpu.VMEM((N_DEV, BLK, D_MODEL), jnp.float32),
            pltpu.SemaphoreType.DMA((N_HOP,)),
            pltpu.SemaphoreType.DMA((N_HOP,)),
            pltpu.SemaphoreType.DMA((N_HOP,)),
            pltpu.SemaphoreType.DMA((N_HOP,)),
        ],
        compiler_params=pltpu.CompilerParams(
            collective_id=0,
            dimension_semantics=("arbitrary",),
        ),
    )(x, slot.reshape(1, N_TOK), slot.reshape(N_TOK, 1), expert_W)


# device time: 51252 ns/iter; 4.0205x vs baseline; 1.5707x over previous
import jax
import jax.numpy as jnp
from jax import lax
from jax.experimental import pallas as pl
from jax.experimental.pallas import tpu as pltpu

N_DEV = 4
N_TOK = 2048
D_MODEL = 1024
N_EXP = 32
N_LOCAL = 8
CAPACITY = 51
CAP_PAD = 64
BLK = N_LOCAL * CAP_PAD
HALF = BLK // 2
N_HOP = N_DEV - 1
SENTINEL = 1 << 20


def _body(x_ref, route_row_ref, ew_ref, out_ref,
          blocks_ref, slot_row_ref, cw_send, ccw_send, cw_recv, ccw_recv):
    k = pl.program_id(0)
    my = lax.axis_index("i")
    left = lax.rem(my + N_DEV - 1, N_DEV)
    right = lax.rem(my + 1, N_DEV)

    def cw_copy(h):
        idx = lax.rem(my - h + N_DEV, N_DEV)
        return pltpu.make_async_remote_copy(
            src_ref=blocks_ref.at[idx, pl.ds(0, HALF), :],
            dst_ref=blocks_ref.at[idx, pl.ds(0, HALF), :],
            send_sem=cw_send.at[h],
            recv_sem=cw_recv.at[h],
            device_id=(right,),
            device_id_type=pl.DeviceIdType.MESH,
        )

    def ccw_copy(h):
        idx = lax.rem(my + h, N_DEV)
        return pltpu.make_async_remote_copy(
            src_ref=blocks_ref.at[idx, pl.ds(HALF, HALF), :],
            dst_ref=blocks_ref.at[idx, pl.ds(HALF, HALF), :],
            send_sem=ccw_send.at[h],
            recv_sem=ccw_recv.at[h],
            device_id=(left,),
            device_id_type=pl.DeviceIdType.MESH,
        )

    @pl.when(k == 0)
    def _():
        route = route_row_ref[...]
        iota_e = lax.broadcasted_iota(jnp.int32, (N_EXP, 1), 0)
        oh = (route == iota_e).astype(jnp.int32)
        pos = oh
        sh = 1
        while sh < N_TOK:
            shifted = jnp.concatenate(
                [jnp.zeros((N_EXP, sh), jnp.int32), pos[:, : N_TOK - sh]],
                axis=1,
            )
            pos = pos + shifted
            sh *= 2
        pos_tok = jnp.sum(oh * pos, axis=0, keepdims=True)
        slot = jnp.where(
            pos_tok <= CAPACITY, route * CAP_PAD + pos_tok - 1, SENTINEL
        )
        slot_row_ref[...] = slot

    base = my * BLK + k * CAP_PAD
    iota_s = lax.broadcasted_iota(jnp.int32, (CAP_PAD, 1), 0)
    dt = (slot_row_ref[...] == iota_s + base).astype(jnp.float32)
    compact_x = jnp.dot(dt, x_ref[...], preferred_element_type=jnp.float32)
    y = jnp.dot(compact_x, ew_ref[0], preferred_element_type=jnp.float32)
    blocks_ref[my, pl.ds(k * CAP_PAD, CAP_PAD), :] = y.astype(jnp.bfloat16)

    @pl.when(k == N_LOCAL // 2 - 1)
    def _():
        barrier_sem = pltpu.get_barrier_semaphore()
        for nbr in (left, right):
            pl.semaphore_signal(
                barrier_sem, inc=1,
                device_id=(nbr,), device_id_type=pl.DeviceIdType.MESH,
            )
        pl.semaphore_wait(barrier_sem, 2)
        cw_copy(0).start()

    @pl.when(k == N_LOCAL - 1)
    def _():
        def scatter(blk_idx, half, init):
            off = blk_idx * BLK + half * HALF
            iota_c = lax.broadcasted_iota(jnp.int32, (HALF, 1), 0)
            d_t = (slot_row_ref[...] == iota_c + off).astype(jnp.bfloat16)
            contrib = lax.dot_general(
                d_t,
                blocks_ref[blk_idx, pl.ds(half * HALF, HALF), :],
                ((( 0,), (0,)), ((), ())),
                preferred_element_type=jnp.float32,
            )
            if init:
                out_ref[...] = contrib
            else:
                out_ref[...] += contrib

        ccw_copy(0).start()
        cw_copy(0).wait()
        cw_copy(1).start()
        scatter(my, 0, init=True)
        scatter(my, 1, init=False)
        ccw_copy(0).wait()
        cw_copy(1).wait()
        ccw_copy(1).start()
        cw_copy(2).start()
        scatter(lax.rem(my - 1 + N_DEV, N_DEV), 0, init=False)
        scatter(lax.rem(my + 1, N_DEV), 1, init=False)
        ccw_copy(1).wait()
        cw_copy(2).wait()
        ccw_copy(2).start()
        scatter(lax.rem(my - 2 + N_DEV, N_DEV), 0, init=False)
        scatter(lax.rem(my + 2, N_DEV), 1, init=False)
        ccw_copy(2).wait()
        scatter(lax.rem(my + 1, N_DEV), 0, init=False)
        scatter(lax.rem(my - 1 + N_DEV, N_DEV), 1, init=False)


def kernel(x, router_W, route_idx, expert_W):
    del router_W

    return pl.pallas_call(
        _body,
        grid=(N_LOCAL,),
        in_specs=[
            pl.BlockSpec((N_TOK, D_MODEL), lambda k: (0, 0)),
            pl.BlockSpec((1, N_TOK), lambda k: (0, 0)),
            pl.BlockSpec((1, D_MODEL, D_MODEL), lambda k: (k, 0, 0)),
        ],
        out_specs=pl.BlockSpec((N_TOK, D_MODEL), lambda k: (0, 0)),
        out_shape=jax.ShapeDtypeStruct((N_TOK, D_MODEL), jnp.float32),
        scratch_shapes=[
            pltpu.VMEM((N_DEV, BLK, D_MODEL), jnp.bfloat16),
            pltpu.VMEM((1, N_TOK), jnp.int32),
            pltpu.SemaphoreType.DMA((N_HOP,)),
            pltpu.SemaphoreType.DMA((N_HOP,)),
            pltpu.SemaphoreType.DMA((N_HOP,)),
            pltpu.SemaphoreType.DMA((N_HOP,)),
        ],
        compiler_params=pltpu.CompilerParams(
            collective_id=0,
            dimension_semantics=("arbitrary",),
        ),
    )(x, route_idx.reshape(1, N_TOK), expert_W)


# device time: 40454 ns/iter; 5.0936x vs baseline; 1.2669x over previous
import jax
import jax.numpy as jnp
from jax import lax
from jax.experimental import pallas as pl
from jax.experimental.pallas import tpu as pltpu

N_DEV = 4
N_TOK = 2048
D_MODEL = 1024
N_EXP = 32
N_LOCAL = 8
CAPACITY = 51
CAP_PAD = 64
BLK = N_LOCAL * CAP_PAD
SENTINEL = 1 << 20


def _body(x_ref, route_row_ref, ew_ref, out_ref,
          blocks_ref, slot_row_ref,
          snd_r, snd_l, fwd_snd, rcv_l, rcv_r, fwd_rcv):
    k = pl.program_id(0)
    my = lax.axis_index("i")
    left = lax.rem(my + N_DEV - 1, N_DEV)
    right = lax.rem(my + 1, N_DEV)

    def tile_copy(owner, j, send_sem, recv_sem, target):
        return pltpu.make_async_remote_copy(
            src_ref=blocks_ref.at[owner, pl.ds(j * CAP_PAD, CAP_PAD), :],
            dst_ref=blocks_ref.at[owner, pl.ds(j * CAP_PAD, CAP_PAD), :],
            send_sem=send_sem,
            recv_sem=recv_sem,
            device_id=(target,),
            device_id_type=pl.DeviceIdType.MESH,
        )

    @pl.when(k == 0)
    def _():
        barrier_sem = pltpu.get_barrier_semaphore()
        for nbr in (left, right):
            pl.semaphore_signal(
                barrier_sem, inc=1,
                device_id=(nbr,), device_id_type=pl.DeviceIdType.MESH,
            )
        pl.semaphore_wait(barrier_sem, 2)

        route = route_row_ref[...]
        iota_e = lax.broadcasted_iota(jnp.int32, (N_EXP, 1), 0)
        oh = (route == iota_e).astype(jnp.int32)
        pos = oh
        sh = 1
        while sh < N_TOK:
            shifted = jnp.concatenate(
                [jnp.zeros((N_EXP, sh), jnp.int32), pos[:, : N_TOK - sh]],
                axis=1,
            )
            pos = pos + shifted
            sh *= 2
        pos_tok = jnp.sum(oh * pos, axis=0, keepdims=True)
        slot_row_ref[...] = jnp.where(
            pos_tok <= CAPACITY, route * CAP_PAD + pos_tok - 1, SENTINEL
        )

    base = my * BLK + k * CAP_PAD
    iota_s = lax.broadcasted_iota(jnp.int32, (CAP_PAD, 1), 0)
    dt = (slot_row_ref[...] == iota_s + base).astype(jnp.float32)
    compact_x = jnp.dot(dt, x_ref[...], preferred_element_type=jnp.float32)
    y = jnp.dot(compact_x, ew_ref[0], preferred_element_type=jnp.float32)
    blocks_ref[my, pl.ds(k * CAP_PAD, CAP_PAD), :] = y.astype(jnp.bfloat16)

    tile_copy(my, k, snd_r.at[k], rcv_l.at[k], right).start()
    tile_copy(my, k, snd_l.at[k], rcv_r.at[k], left).start()

    @pl.when((k >= 2) & (k % 2 == 0))
    def _():
        j = k - 2
        tile_copy(left, j, snd_r.at[j], rcv_l.at[j], right).wait_recv()
        tile_copy(left, j, fwd_snd.at[j], fwd_rcv.at[j], right).start()

    @pl.when((k >= 2) & (k % 2 == 1))
    def _():
        j = k - 2
        tile_copy(right, j, snd_l.at[j], rcv_r.at[j], left).wait_recv()
        tile_copy(right, j, fwd_snd.at[j], fwd_rcv.at[j], left).start()

    @pl.when(k == N_LOCAL - 1)
    def _():
        def scatter(blk_idx, init):
            iota_c = lax.broadcasted_iota(jnp.int32, (BLK, 1), 0)
            d_t = (slot_row_ref[...] == iota_c + blk_idx * BLK)
            contrib = lax.dot_general(
                d_t.astype(jnp.bfloat16),
                blocks_ref[blk_idx],
                (((0,), (0,)), ((), ())),
                preferred_element_type=jnp.float32,
            )
            if init:
                out_ref[...] = contrib
            else:
                out_ref[...] += contrib

        tile_copy(left, 6, snd_r.at[6], rcv_l.at[6], right).wait_recv()
        tile_copy(left, 6, fwd_snd.at[6], fwd_rcv.at[6], right).start()
        scatter(my, init=True)
        tile_copy(right, 7, snd_l.at[7], rcv_r.at[7], left).wait_recv()
        tile_copy(right, 7, fwd_snd.at[7], fwd_rcv.at[7], left).start()

        for j in (1, 3, 5, 7):
            tile_copy(left, j, snd_r.at[j], rcv_l.at[j], right).wait_recv()
        scatter(left, init=False)
        for j in (0, 2, 4, 6):
            tile_copy(right, j, snd_l.at[j], rcv_r.at[j], left).wait_recv()
        scatter(right, init=False)
        diag = lax.rem(my + 2, N_DEV)
        for j in range(N_LOCAL):
            src = left if j % 2 == 0 else right
            tgt = right if j % 2 == 0 else left
            tile_copy(src, j, fwd_snd.at[j], fwd_rcv.at[j], tgt).wait_recv()
        scatter(diag, init=False)

        for j in range(N_LOCAL):
            tile_copy(my, j, snd_r.at[j], rcv_l.at[j], right).wait_send()
            tile_copy(my, j, snd_l.at[j], rcv_r.at[j], left).wait_send()
            src = left if j % 2 == 0 else right
            tgt = right if j % 2 == 0 else left
            tile_copy(src, j, fwd_snd.at[j], fwd_rcv.at[j], tgt).wait_send()


def kernel(x, router_W, route_idx, expert_W):
    del router_W

    return pl.pallas_call(
        _body,
        grid=(N_LOCAL,),
        in_specs=[
            pl.BlockSpec((N_TOK, D_MODEL), lambda k: (0, 0)),
            pl.BlockSpec((1, N_TOK), lambda k: (0, 0)),
            pl.BlockSpec((1, D_MODEL, D_MODEL), lambda k: (k, 0, 0)),
        ],
        out_specs=pl.BlockSpec((N_TOK, D_MODEL), lambda k: (0, 0)),
        out_shape=jax.ShapeDtypeStruct((N_TOK, D_MODEL), jnp.float32),
        scratch_shapes=[
            pltpu.VMEM((N_DEV, BLK, D_MODEL), jnp.bfloat16),
            pltpu.VMEM((1, N_TOK), jnp.int32),
            pltpu.SemaphoreType.DMA((N_LOCAL,)),
            pltpu.SemaphoreType.DMA((N_LOCAL,)),
            pltpu.SemaphoreType.DMA((N_LOCAL,)),
            pltpu.SemaphoreType.DMA((N_LOCAL,)),
            pltpu.SemaphoreType.DMA((N_LOCAL,)),
            pltpu.SemaphoreType.DMA((N_LOCAL,)),
        ],
        compiler_params=pltpu.CompilerParams(
            collective_id=0,
            dimension_semantics=("arbitrary",),
        ),
    )(x, route_idx.reshape(1, N_TOK), expert_W)


# device time: 40260 ns/iter; 5.1182x vs baseline; 1.0048x over previous
import jax
import jax.numpy as jnp
from jax import lax
from jax.experimental import pallas as pl
from jax.experimental.pallas import tpu as pltpu

N_DEV = 4
N_TOK = 2048
D_MODEL = 1024
N_EXP = 32
N_LOCAL = 8
CAPACITY = 51
CAP_PAD = 64
BLK = N_LOCAL * CAP_PAD
SENTINEL = 1 << 20


def _body(x_ref, route_row_ref, ew_ref, out_ref,
          blocks_ref, slot_row_ref, cx_ref,
          snd_r, snd_l, fwd_snd, rcv_l, rcv_r, fwd_rcv):
    k = pl.program_id(0)
    my = lax.axis_index("i")
    left = lax.rem(my + N_DEV - 1, N_DEV)
    right = lax.rem(my + 1, N_DEV)

    def tile_copy(owner, j, send_sem, recv_sem, target):
        return pltpu.make_async_remote_copy(
            src_ref=blocks_ref.at[owner, pl.ds(j * CAP_PAD, CAP_PAD), :],
            dst_ref=blocks_ref.at[owner, pl.ds(j * CAP_PAD, CAP_PAD), :],
            send_sem=send_sem,
            recv_sem=recv_sem,
            device_id=(target,),
            device_id_type=pl.DeviceIdType.MESH,
        )

    @pl.when(k == 0)
    def _():
        barrier_sem = pltpu.get_barrier_semaphore()
        for nbr in (left, right):
            pl.semaphore_signal(
                barrier_sem, inc=1,
                device_id=(nbr,), device_id_type=pl.DeviceIdType.MESH,
            )
        pl.semaphore_wait(barrier_sem, 2)

        route = route_row_ref[...]
        iota_e = lax.broadcasted_iota(jnp.int32, (N_EXP, 1), 0)
        oh = (route == iota_e).astype(jnp.int32)
        pos = oh
        sh = 1
        while sh < N_TOK:
            shifted = jnp.concatenate(
                [jnp.zeros((N_EXP, sh), jnp.int32), pos[:, : N_TOK - sh]],
                axis=1,
            )
            pos = pos + shifted
            sh *= 2
        pos_tok = jnp.sum(oh * pos, axis=0, keepdims=True)
        slot_row_ref[...] = jnp.where(
            pos_tok <= CAPACITY, route * CAP_PAD + pos_tok - 1, SENTINEL
        )

        iota_s = lax.broadcasted_iota(jnp.int32, (BLK, 1), 0)
        dt = (slot_row_ref[...] == iota_s + my * BLK).astype(jnp.float32)
        cx_ref[...] = jnp.dot(
            dt, x_ref[...], preferred_element_type=jnp.float32
        )

    y = jnp.dot(
        cx_ref[pl.ds(k * CAP_PAD, CAP_PAD), :],
        ew_ref[0],
        preferred_element_type=jnp.float32,
    )
    blocks_ref[my, pl.ds(k * CAP_PAD, CAP_PAD), :] = y.astype(jnp.bfloat16)

    tile_copy(my, k, snd_r.at[k], rcv_l.at[k], right).start()
    tile_copy(my, k, snd_l.at[k], rcv_r.at[k], left).start()

    @pl.when((k >= 2) & (k % 2 == 0))
    def _():
        j = k - 2
        tile_copy(left, j, snd_r.at[j], rcv_l.at[j], right).wait_recv()
        tile_copy(left, j, fwd_snd.at[j], fwd_rcv.at[j], right).start()

    @pl.when((k >= 2) & (k % 2 == 1))
    def _():
        j = k - 2
        tile_copy(right, j, snd_l.at[j], rcv_r.at[j], left).wait_recv()
        tile_copy(right, j, fwd_snd.at[j], fwd_rcv.at[j], left).start()

    @pl.when(k == N_LOCAL - 1)
    def _():
        def scatter(blk_idx, init):
            iota_c = lax.broadcasted_iota(jnp.int32, (BLK, 1), 0)
            d_t = (slot_row_ref[...] == iota_c + blk_idx * BLK)
            contrib = lax.dot_general(
                d_t.astype(jnp.bfloat16),
                blocks_ref[blk_idx],
                (((0,), (0,)), ((), ())),
                preferred_element_type=jnp.float32,
            )
            if init:
                out_ref[...] = contrib
            else:
                out_ref[...] += contrib

        tile_copy(left, 6, snd_r.at[6], rcv_l.at[6], right).wait_recv()
        tile_copy(left, 6, fwd_snd.at[6], fwd_rcv.at[6], right).start()
        scatter(my, init=True)
        tile_copy(right, 7, snd_l.at[7], rcv_r.at[7], left).wait_recv()
        tile_copy(right, 7, fwd_snd.at[7], fwd_rcv.at[7], left).start()

        for j in (1, 3, 5, 7):
            tile_copy(left, j, snd_r.at[j], rcv_l.at[j], right).wait_recv()
        scatter(left, init=False)
        for j in (0, 2, 4, 6):
            tile_copy(right, j, snd_l.at[j], rcv_r.at[j], left).wait_recv()
        scatter(right, init=False)
        diag = lax.rem(my + 2, N_DEV)
        for j in range(N_LOCAL):
            src = left if j % 2 == 0 else right
            tgt = right if j % 2 == 0 else left
            tile_copy(src, j, fwd_snd.at[j], fwd_rcv.at[j], tgt).wait_recv()
        scatter(diag, init=False)

        for j in range(N_LOCAL):
            tile_copy(my, j, snd_r.at[j], rcv_l.at[j], right).wait_send()
            tile_copy(my, j, snd_l.at[j], rcv_r.at[j], left).wait_send()
            src = left if j % 2 == 0 else right
            tgt = right if j % 2 == 0 else left
            tile_copy(src, j, fwd_snd.at[j], fwd_rcv.at[j], tgt).wait_send()


def kernel(x, router_W, route_idx, expert_W):
    del router_W

    return pl.pallas_call(
        _body,
        grid=(N_LOCAL,),
        in_specs=[
            pl.BlockSpec((N_TOK, D_MODEL), lambda k: (0, 0)),
            pl.BlockSpec((1, N_TOK), lambda k: (0, 0)),
            pl.BlockSpec((1, D_MODEL, D_MODEL), lambda k: (k, 0, 0)),
        ],
        out_specs=pl.BlockSpec((N_TOK, D_MODEL), lambda k: (0, 0)),
        out_shape=jax.ShapeDtypeStruct((N_TOK, D_MODEL), jnp.float32),
        scratch_shapes=[
            pltpu.VMEM((N_DEV, BLK, D_MODEL), jnp.bfloat16),
            pltpu.VMEM((1, N_TOK), jnp.int32),
            pltpu.VMEM((BLK, D_MODEL), jnp.float32),
            pltpu.SemaphoreType.DMA((N_LOCAL,)),
            pltpu.SemaphoreType.DMA((N_LOCAL,)),
            pltpu.SemaphoreType.DMA((N_LOCAL,)),
            pltpu.SemaphoreType.DMA((N_LOCAL,)),
            pltpu.SemaphoreType.DMA((N_LOCAL,)),
            pltpu.SemaphoreType.DMA((N_LOCAL,)),
        ],
        compiler_params=pltpu.CompilerParams(
            collective_id=0,
            dimension_semantics=("arbitrary",),
        ),
    )(x, route_idx.reshape(1, N_TOK), expert_W)
